# baseline (speedup 1.0000x reference)
import functools

import jax
import jax.numpy as jnp
from jax.experimental import pallas as pl
from jax.experimental.pallas import tpu as pltpu

_BINS = 30
_BM = 512
_P = 2


def _ghm_block(pred_ref, tgt_ref, w_ref, out_ref, acc_ref, *, inner):
    i = pl.program_id(1)

    @pl.when(i == 0)
    def _init():
        for k in range(_BINS):
            acc_ref[0, k] = 0.0
            acc_ref[1, k] = 0.0
        acc_ref[2, 0] = 0.0
        acc_ref[2, 1] = 0.0

    p = pred_ref[...]
    t = tgt_ref[...]
    w = w_ref[...]

    col = jax.lax.broadcasted_iota(jnp.int32, p.shape, 1)
    onehot = col == t

    ap = jnp.abs(p)
    e = jnp.exp(-ap)
    r = 1.0 / (1.0 + e)
    s = jnp.where(p >= 0, r, e * r)
    g = jnp.where(onehot, 1.0 - s, s)
    loss = jnp.maximum(p, 0.0) + jnp.log1p(e) - jnp.where(onehot, p, 0.0)

    w_row = jnp.sum(jnp.where(onehot, w, 0.0), axis=1, keepdims=True)
    val = loss * w_row

    u = g * _BINS

    for k in range(1, _BINS):
        m = u < float(k)
        acc_ref[0, k] += jnp.sum(jnp.where(m, 1.0, 0.0))
        acc_ref[1, k] += jnp.sum(jnp.where(m, val, 0.0))
    acc_ref[2, 0] += jnp.sum(w_row)
    acc_ref[2, 1] += jnp.sum(val)

    @pl.when(i == inner - 1)
    def _fin():
        for k in range(_BINS):
            out_ref[0, 0, k] = acc_ref[0, k]
            out_ref[0, 0, _BINS + k] = acc_ref[1, k]
        out_ref[0, 0, 2 * _BINS] = acc_ref[2, 0]
        out_ref[0, 0, 2 * _BINS + 1] = acc_ref[2, 1]


def _combine(part_ref, out_ref, *, tot, C):
    n_elems = jnp.float32(tot)
    total = jnp.float32(0.0)
    n = jnp.float32(0.0)
    cum_c = [jnp.float32(0.0)] * (_BINS + 1)
    cum_s = [jnp.float32(0.0)] * (_BINS + 1)
    wsum = jnp.float32(0.0)
    stot = jnp.float32(0.0)
    for c in range(_P):
        for k in range(1, _BINS):
            cum_c[k] += part_ref[c, 0, k]
            cum_s[k] += part_ref[c, 0, _BINS + k]
        wsum += part_ref[c, 0, 2 * _BINS]
        stot += part_ref[c, 0, 2 * _BINS + 1]
    cum_c[_BINS] = n_elems
    cum_s[_BINS] = stot
    for b in range(_BINS):
        cnt = cum_c[b + 1] - cum_c[b]
        n += jnp.where(cnt > 0.0, 1.0, 0.0)
        total += (cum_s[b + 1] - cum_s[b]) / jnp.maximum(cnt, 1.0)
    out_ref[0, 0] = (tot / n) * total / (wsum * C)


def kernel(pred, target, weight):
    B, C = pred.shape
    nblk = B // _BM
    inner = nblk // _P
    t2 = target.reshape(B, 1)
    w2 = weight.reshape(1, C)
    part = pl.pallas_call(
        functools.partial(_ghm_block, inner=inner),
        grid=(_P, inner),
        in_specs=[
            pl.BlockSpec((_BM, C), lambda c, i: (c * inner + i, 0)),
            pl.BlockSpec((_BM, 1), lambda c, i: (c * inner + i, 0)),
            pl.BlockSpec((1, C), lambda c, i: (0, 0)),
        ],
        out_specs=pl.BlockSpec((1, 1, 2 * _BINS + 2), lambda c, i: (c, 0, 0), memory_space=pltpu.SMEM),
        out_shape=jax.ShapeDtypeStruct((_P, 1, 2 * _BINS + 2), jnp.float32),
        scratch_shapes=[pltpu.SMEM((4, _BINS + 2), jnp.float32)],
        compiler_params=pltpu.CompilerParams(
            dimension_semantics=("parallel", "arbitrary")
        ),
    )(pred, t2, w2)
    out = pl.pallas_call(
        functools.partial(_combine, tot=float(B * C), C=C),
        in_specs=[pl.BlockSpec(memory_space=pltpu.SMEM)],
        out_specs=pl.BlockSpec(memory_space=pltpu.SMEM),
        out_shape=jax.ShapeDtypeStruct((1, 1), jnp.float32),
    )(part)
    return out[0, 0]


# speedup vs baseline: 1.1535x; 1.1535x over previous
import functools

import jax
import jax.numpy as jnp
from jax.experimental import pallas as pl
from jax.experimental.pallas import tpu as pltpu

_BINS = 30
_BM = 512
_CH = 8
_LANES = 1024
_GROUPS = [list(range(1, 9)), list(range(9, 17)), list(range(17, 25)),
           list(range(25, 30))]


def _ghm_block(pred_ref, tgt_ref, w_ref, out_ref, u_scr, v_scr, acc_ref, *, tot):
    i = pl.program_id(0)
    nblk = pl.num_programs(0)

    @pl.when(i == 0)
    def _init():
        for k in range(_BINS):
            acc_ref[0, k] = 0.0
            acc_ref[1, k] = 0.0
        acc_ref[2, 0] = 0.0
        acc_ref[2, 1] = 0.0

    p = pred_ref[...]
    t = tgt_ref[...]
    w = w_ref[...]
    C = p.shape[1]

    col = jax.lax.broadcasted_iota(jnp.int32, p.shape, 1)
    onehot = col == t

    ap = jnp.abs(p)
    e = jnp.exp(-ap)
    r = 1.0 / (1.0 + e)
    s = jnp.where(p >= 0, r, e * r)
    g = jnp.where(onehot, 1.0 - s, s)
    loss = jnp.maximum(p, 0.0) + jnp.log1p(e) - jnp.where(onehot, p, 0.0)

    w_row = jnp.sum(jnp.where(onehot, w, 0.0), axis=1, keepdims=True)
    val = loss * w_row
    u = g * _BINS

    pad = _LANES - C
    u_scr[...] = jnp.concatenate(
        [u, jnp.full((p.shape[0], pad), 1e9, jnp.float32)], axis=1)
    v_scr[...] = jnp.concatenate(
        [val, jnp.zeros((p.shape[0], pad), jnp.float32)], axis=1)

    acc_ref[2, 0] += jnp.sum(w_row)
    acc_ref[2, 1] += jnp.sum(val)

    nchunk = p.shape[0] // _CH
    for ks in _GROUPS:
        zero = jnp.zeros((_CH, 128), jnp.float32)
        init = tuple(zero for _ in range(2 * len(ks)))

        def body(c, carry, ks=ks):
            accs = list(carry)
            uc = u_scr[pl.ds(c * _CH, _CH), :]
            vc = v_scr[pl.ds(c * _CH, _CH), :]
            for lg in range(_LANES // 128):
                sl = slice(lg * 128, (lg + 1) * 128)
                u_sl = uc[:, sl]
                v_sl = vc[:, sl]
                for j, k in enumerate(ks):
                    m = u_sl < float(k)
                    accs[2 * j] = accs[2 * j] + jnp.where(m, 1.0, 0.0)
                    accs[2 * j + 1] = accs[2 * j + 1] + jnp.where(m, v_sl, 0.0)
            return tuple(accs)

        res = jax.lax.fori_loop(0, nchunk, body, init)
        for j, k in enumerate(ks):
            acc_ref[0, k] += jnp.sum(res[2 * j])
            acc_ref[1, k] += jnp.sum(res[2 * j + 1])

    @pl.when(i == nblk - 1)
    def _fin():
        n_elems = jnp.float32(tot)
        total = jnp.float32(0.0)
        n = jnp.float32(0.0)
        for b in range(_BINS):
            c_lo = acc_ref[0, b] if b > 0 else jnp.float32(0.0)
            c_hi = acc_ref[0, b + 1] if b + 1 < _BINS else n_elems
            s_lo = acc_ref[1, b] if b > 0 else jnp.float32(0.0)
            s_hi = acc_ref[1, b + 1] if b + 1 < _BINS else acc_ref[2, 1]
            cnt = c_hi - c_lo
            n += jnp.where(cnt > 0.0, 1.0, 0.0)
            total += (s_hi - s_lo) / jnp.maximum(cnt, 1.0)
        wsum = acc_ref[2, 0] * C
        out_ref[0, 0] = (tot / n) * total / wsum


def kernel(pred, target, weight):
    B, C = pred.shape
    nblk = B // _BM
    t2 = target.reshape(B, 1)
    w2 = weight.reshape(1, C)
    out = pl.pallas_call(
        functools.partial(_ghm_block, tot=float(B * C)),
        grid=(nblk,),
        in_specs=[
            pl.BlockSpec((_BM, C), lambda i: (i, 0)),
            pl.BlockSpec((_BM, 1), lambda i: (i, 0)),
            pl.BlockSpec((1, C), lambda i: (0, 0)),
        ],
        out_specs=pl.BlockSpec(memory_space=pltpu.SMEM),
        out_shape=jax.ShapeDtypeStruct((1, 1), jnp.float32),
        scratch_shapes=[
            pltpu.VMEM((_BM, _LANES), jnp.float32),
            pltpu.VMEM((_BM, _LANES), jnp.float32),
            pltpu.SMEM((4, _BINS + 2), jnp.float32),
        ],
    )(pred, t2, w2)
    return out[0, 0]


# speedup vs baseline: 1.2238x; 1.0609x over previous
import functools

import jax
import jax.numpy as jnp
from jax.experimental import pallas as pl
from jax.experimental.pallas import tpu as pltpu

_BINS = 30
_BM = 512
_CH = 8
_LANES = 1024
_GROUPS = [list(range(1, 30))]


def _ghm_block(pred_ref, tgt_ref, w_ref, out_ref, u_scr, v_scr, acc_ref, *, tot):
    i = pl.program_id(0)
    nblk = pl.num_programs(0)

    @pl.when(i == 0)
    def _init():
        for k in range(_BINS):
            acc_ref[0, k] = 0.0
            acc_ref[1, k] = 0.0
        acc_ref[2, 0] = 0.0
        acc_ref[2, 1] = 0.0

    p = pred_ref[...]
    t = tgt_ref[...]
    w = w_ref[...]
    C = p.shape[1]

    col = jax.lax.broadcasted_iota(jnp.int32, p.shape, 1)
    onehot = col == t

    ap = jnp.abs(p)
    e = jnp.exp(-ap)
    r = 1.0 / (1.0 + e)
    s = jnp.where(p >= 0, r, e * r)
    g = jnp.where(onehot, 1.0 - s, s)
    loss = jnp.maximum(p, 0.0) + jnp.log1p(e) - jnp.where(onehot, p, 0.0)

    w_row = jnp.sum(jnp.where(onehot, w, 0.0), axis=1, keepdims=True)
    val = loss * w_row
    u = g * _BINS

    pad = _LANES - C
    u_scr[...] = jnp.concatenate(
        [u, jnp.full((p.shape[0], pad), 1e9, jnp.float32)], axis=1)
    v_scr[...] = jnp.concatenate(
        [val, jnp.zeros((p.shape[0], pad), jnp.float32)], axis=1)

    acc_ref[2, 0] += jnp.sum(w_row)
    acc_ref[2, 1] += jnp.sum(val)

    nchunk = p.shape[0] // _CH
    for ks in _GROUPS:
        zero = jnp.zeros((_CH, 128), jnp.float32)
        init = tuple(zero for _ in range(2 * len(ks)))

        def body(c, carry, ks=ks):
            accs = list(carry)
            uc = u_scr[pl.ds(c * _CH, _CH), :]
            vc = v_scr[pl.ds(c * _CH, _CH), :]
            for lg in range(_LANES // 128):
                sl = slice(lg * 128, (lg + 1) * 128)
                u_sl = uc[:, sl]
                v_sl = vc[:, sl]
                for j, k in enumerate(ks):
                    m = u_sl < float(k)
                    accs[2 * j] = accs[2 * j] + jnp.where(m, 1.0, 0.0)
                    accs[2 * j + 1] = accs[2 * j + 1] + jnp.where(m, v_sl, 0.0)
            return tuple(accs)

        res = jax.lax.fori_loop(0, nchunk, body, init)
        for j, k in enumerate(ks):
            acc_ref[0, k] += jnp.sum(res[2 * j])
            acc_ref[1, k] += jnp.sum(res[2 * j + 1])

    @pl.when(i == nblk - 1)
    def _fin():
        n_elems = jnp.float32(tot)
        total = jnp.float32(0.0)
        n = jnp.float32(0.0)
        for b in range(_BINS):
            c_lo = acc_ref[0, b] if b > 0 else jnp.float32(0.0)
            c_hi = acc_ref[0, b + 1] if b + 1 < _BINS else n_elems
            s_lo = acc_ref[1, b] if b > 0 else jnp.float32(0.0)
            s_hi = acc_ref[1, b + 1] if b + 1 < _BINS else acc_ref[2, 1]
            cnt = c_hi - c_lo
            n += jnp.where(cnt > 0.0, 1.0, 0.0)
            total += (s_hi - s_lo) / jnp.maximum(cnt, 1.0)
        wsum = acc_ref[2, 0] * C
        out_ref[0, 0] = (tot / n) * total / wsum


def kernel(pred, target, weight):
    B, C = pred.shape
    nblk = B // _BM
    t2 = target.reshape(B, 1)
    w2 = weight.reshape(1, C)
    out = pl.pallas_call(
        functools.partial(_ghm_block, tot=float(B * C)),
        grid=(nblk,),
        in_specs=[
            pl.BlockSpec((_BM, C), lambda i: (i, 0)),
            pl.BlockSpec((_BM, 1), lambda i: (i, 0)),
            pl.BlockSpec((1, C), lambda i: (0, 0)),
        ],
        out_specs=pl.BlockSpec(memory_space=pltpu.SMEM),
        out_shape=jax.ShapeDtypeStruct((1, 1), jnp.float32),
        scratch_shapes=[
            pltpu.VMEM((_BM, _LANES), jnp.float32),
            pltpu.VMEM((_BM, _LANES), jnp.float32),
            pltpu.SMEM((4, _BINS + 2), jnp.float32),
        ],
    )(pred, t2, w2)
    return out[0, 0]


# speedup vs baseline: 1.2544x; 1.0250x over previous
import functools

import jax
import jax.numpy as jnp
from jax.experimental import pallas as pl
from jax.experimental.pallas import tpu as pltpu

_BINS = 30
_BM = 512
_CH = 16
_LANES = 1024
_GROUPS = [list(range(1, 30))]


def _ghm_block(pred_ref, tgt_ref, w_ref, out_ref, u_scr, v_scr, acc_ref, *, tot):
    i = pl.program_id(0)
    nblk = pl.num_programs(0)

    @pl.when(i == 0)
    def _init():
        for k in range(_BINS):
            acc_ref[0, k] = 0.0
            acc_ref[1, k] = 0.0
        acc_ref[2, 0] = 0.0
        acc_ref[2, 1] = 0.0

    p = pred_ref[...]
    t = tgt_ref[...]
    w = w_ref[...]
    C = p.shape[1]

    col = jax.lax.broadcasted_iota(jnp.int32, p.shape, 1)
    onehot = col == t

    ap = jnp.abs(p)
    e = jnp.exp(-ap)
    r = 1.0 / (1.0 + e)
    s = jnp.where(p >= 0, r, e * r)
    g = jnp.where(onehot, 1.0 - s, s)
    loss = jnp.maximum(p, 0.0) + jnp.log1p(e) - jnp.where(onehot, p, 0.0)

    w_row = jnp.sum(jnp.where(onehot, w, 0.0), axis=1, keepdims=True)
    val = loss * w_row
    u = g * _BINS

    pad = _LANES - C
    u_scr[...] = jnp.concatenate(
        [u, jnp.full((p.shape[0], pad), 1e9, jnp.float32)], axis=1)
    v_scr[...] = jnp.concatenate(
        [val, jnp.zeros((p.shape[0], pad), jnp.float32)], axis=1)

    acc_ref[2, 0] += jnp.sum(w_row)
    acc_ref[2, 1] += jnp.sum(val)

    nchunk = p.shape[0] // _CH
    for ks in _GROUPS:
        zero = jnp.zeros((8, 128), jnp.float32)
        init = tuple(zero for _ in range(2 * len(ks)))

        def body(c, carry, ks=ks):
            accs = list(carry)
            uc = u_scr[pl.ds(c * _CH, _CH), :]
            vc = v_scr[pl.ds(c * _CH, _CH), :]
            for sr in range(_CH // 8):
                rs = slice(sr * 8, (sr + 1) * 8)
                for lg in range(_LANES // 128):
                    sl = slice(lg * 128, (lg + 1) * 128)
                    u_sl = uc[rs, sl]
                    v_sl = vc[rs, sl]
                    for j, k in enumerate(ks):
                        m = u_sl < float(k)
                        accs[2 * j] = accs[2 * j] + jnp.where(m, 1.0, 0.0)
                        accs[2 * j + 1] = accs[2 * j + 1] + jnp.where(m, v_sl, 0.0)
            return tuple(accs)

        res = jax.lax.fori_loop(0, nchunk, body, init)
        for j, k in enumerate(ks):
            acc_ref[0, k] += jnp.sum(res[2 * j])
            acc_ref[1, k] += jnp.sum(res[2 * j + 1])

    @pl.when(i == nblk - 1)
    def _fin():
        n_elems = jnp.float32(tot)
        total = jnp.float32(0.0)
        n = jnp.float32(0.0)
        for b in range(_BINS):
            c_lo = acc_ref[0, b] if b > 0 else jnp.float32(0.0)
            c_hi = acc_ref[0, b + 1] if b + 1 < _BINS else n_elems
            s_lo = acc_ref[1, b] if b > 0 else jnp.float32(0.0)
            s_hi = acc_ref[1, b + 1] if b + 1 < _BINS else acc_ref[2, 1]
            cnt = c_hi - c_lo
            n += jnp.where(cnt > 0.0, 1.0, 0.0)
            total += (s_hi - s_lo) / jnp.maximum(cnt, 1.0)
        wsum = acc_ref[2, 0] * C
        out_ref[0, 0] = (tot / n) * total / wsum


def kernel(pred, target, weight):
    B, C = pred.shape
    nblk = B // _BM
    t2 = target.reshape(B, 1)
    w2 = weight.reshape(1, C)
    out = pl.pallas_call(
        functools.partial(_ghm_block, tot=float(B * C)),
        grid=(nblk,),
        in_specs=[
            pl.BlockSpec((_BM, C), lambda i: (i, 0)),
            pl.BlockSpec((_BM, 1), lambda i: (i, 0)),
            pl.BlockSpec((1, C), lambda i: (0, 0)),
        ],
        out_specs=pl.BlockSpec(memory_space=pltpu.SMEM),
        out_shape=jax.ShapeDtypeStruct((1, 1), jnp.float32),
        scratch_shapes=[
            pltpu.VMEM((_BM, _LANES), jnp.float32),
            pltpu.VMEM((_BM, _LANES), jnp.float32),
            pltpu.SMEM((4, _BINS + 2), jnp.float32),
        ],
    )(pred, t2, w2)
    return out[0, 0]
